# Initial kernel scaffold; baseline (speedup 1.0000x reference)
import jax
import jax.numpy as jnp
from jax.experimental import pallas as pl


def kernel(query_descriptors, test_descriptors, test_labels, test_cameras, query_labels, query_cameras, distractors, junk, maxrank):
    raise NotImplementedError("write your pallas kernel here")


# speedup vs baseline: 47.8385x; 47.8385x over previous
import jax
import jax.numpy as jnp
from jax.experimental import pallas as pl
from jax.experimental.pallas import tpu as pltpu

_Q, _T, _D = 1024, 16384, 256
_MR = 50
_BQ = 8
_MMB = 128


def _dist_kernel(q_ref, t_ref, qrn_ref, trn_ref, o_ref):
    acc = jax.lax.dot_general(
        q_ref[...], t_ref[...], (((1,), (1,)), ((), ())),
        preferred_element_type=jnp.float32,
        precision=jax.lax.Precision.DEFAULT)
    o_ref[...] = 1.0 - acc * qrn_ref[...] * trn_ref[...]


def _sort_kernel(d_ref, meta_ref, ql_ref, qc_ref, o_ref, ap_ref, r_ref, cnt_ref):
    iota = jax.lax.broadcasted_iota(jnp.int32, (_BQ, _T), 1)
    meta = meta_ref[...]
    lab = (meta >> 5) & 127
    cam = (meta >> 2) & 7
    distr = meta & 2
    jnk = meta & 1
    ql = ql_ref[...]
    qc = qc_ref[...]
    lab_eq = lab == ql
    junk2 = (jnk == 1) | (lab_eq & (cam == qc))
    good = (distr == 0) & jnp.logical_not(junk2) & lab_eq

    b = jax.lax.bitcast_convert_type(d_ref[...], jnp.int32)
    s = b ^ jnp.where(b < 0, jnp.int32(0x7FFFFFFF), jnp.int32(0))
    s = (s & jnp.int32(~3)) | jnp.where(good, jnp.int32(2), jnp.int32(0)) \
        | jnp.where(junk2, jnp.int32(1), jnp.int32(0))

    for m in range(1, 15):
        k = 1 << m
        asc = (iota & k) == 0

        def stage(it, s, _k=k, _asc=asc):
            j = jnp.int32(_k) >> (it + 1)
            down = pltpu.roll(s, jnp.int32(_T) - j, 1)
            up = pltpu.roll(s, j, 1)
            lower = (iota & j) == 0
            part = jnp.where(lower, down, up)
            keep = (s < part) == (lower == _asc)
            return jnp.where(keep, s, part)

        s = jax.lax.fori_loop(0, m, stage, s)

    junk_s = s & 1
    good_s = (s >> 1) & 1

    c = (good_s << 16) | junk_s

    def csum(it, c):
        sh = jnp.int32(1) << it
        return c + jnp.where(iota >= sh, pltpu.roll(c, sh, 1), jnp.int32(0))

    c = jax.lax.fori_loop(0, 14, csum, c)
    jc = c & 0xFFFF
    gp = c >> 16

    goodb = good_s == 1
    goodf = jnp.where(goodb, 1.0, 0.0).astype(jnp.float32)
    cnt = jnp.sum(goodf, axis=1, keepdims=True)
    cnt_safe = jnp.maximum(cnt, 1.0)
    terms = gp.astype(jnp.float32) / (iota - jc + 1).astype(jnp.float32) / cnt_safe
    ap_ref[...] = jnp.sum(jnp.where(goodb, terms, 0.0), axis=1, keepdims=True)

    first = goodb & (gp == 1)
    fc = jnp.sum(jnp.where(first, iota, 0), axis=1, keepdims=True)
    fj = jnp.sum(jnp.where(first, jc, 0), axis=1, keepdims=True)
    r_ref[...] = (fc - fj).astype(jnp.float32)
    cnt_ref[...] = cnt

    sc = s & jnp.int32(~3)
    bb = sc ^ jnp.where(sc < 0, jnp.int32(0x7FFFFFFF), jnp.int32(0))
    o_ref[...] = jax.lax.bitcast_convert_type(bb, jnp.float32)


def _fin_kernel(ap_ref, r_ref, cnt_ref, mr_ref, ranks_ref, map_ref):
    r = r_ref[...]
    valid = (cnt_ref[...] > 0.0) & (r < mr_ref[...])
    cols = jax.lax.broadcasted_iota(jnp.int32, (_Q, 64), 1).astype(jnp.float32)
    hits = jnp.where((r == cols) & valid, 1.0, 0.0)
    hist = jnp.sum(hits, axis=0, keepdims=True)
    iota64 = jax.lax.broadcasted_iota(jnp.int32, (1, 64), 1)

    def csum(it, h):
        sh = jnp.int32(1) << it
        return h + jnp.where(iota64 >= sh, pltpu.roll(h, sh, 1), 0.0)

    hist = jax.lax.fori_loop(0, 6, csum, hist)
    ranks_ref[...] = hist * (1.0 / _Q)
    map_ref[...] = jnp.sum(ap_ref[...], axis=0, keepdims=True) * (1.0 / _Q)


def kernel(query_descriptors, test_descriptors, test_labels, test_cameras,
           query_labels, query_cameras, distractors, junk, maxrank):
    qd = query_descriptors.astype(jnp.float32)
    td = test_descriptors.astype(jnp.float32)
    qrn = 1.0 / jnp.sqrt(jnp.sum(qd * qd, axis=1, keepdims=True))
    trn = (1.0 / jnp.sqrt(jnp.sum(td * td, axis=1))).reshape(1, _T)
    meta = ((test_labels.astype(jnp.int32) << 5)
            | (test_cameras.astype(jnp.int32) << 2)
            | (distractors.astype(jnp.int32) << 1)
            | junk.astype(jnp.int32)).reshape(1, _T)

    dists = pl.pallas_call(
        _dist_kernel,
        grid=(_Q // _MMB,),
        in_specs=[
            pl.BlockSpec((_MMB, _D), lambda i: (i, 0)),
            pl.BlockSpec((_T, _D), lambda i: (0, 0)),
            pl.BlockSpec((_MMB, 1), lambda i: (i, 0)),
            pl.BlockSpec((1, _T), lambda i: (0, 0)),
        ],
        out_specs=pl.BlockSpec((_MMB, _T), lambda i: (i, 0)),
        out_shape=jax.ShapeDtypeStruct((_Q, _T), jnp.float32),
    )(qd, td, qrn, trn)

    dsorted, ap, r, cnt = pl.pallas_call(
        _sort_kernel,
        grid=(_Q // _BQ,),
        in_specs=[
            pl.BlockSpec((_BQ, _T), lambda i: (i, 0)),
            pl.BlockSpec((1, _T), lambda i: (0, 0)),
            pl.BlockSpec((_BQ, 1), lambda i: (i, 0)),
            pl.BlockSpec((_BQ, 1), lambda i: (i, 0)),
        ],
        out_specs=[
            pl.BlockSpec((_BQ, _T), lambda i: (i, 0)),
            pl.BlockSpec((_BQ, 1), lambda i: (i, 0)),
            pl.BlockSpec((_BQ, 1), lambda i: (i, 0)),
            pl.BlockSpec((_BQ, 1), lambda i: (i, 0)),
        ],
        out_shape=[
            jax.ShapeDtypeStruct((_Q, _T), jnp.float32),
            jax.ShapeDtypeStruct((_Q, 1), jnp.float32),
            jax.ShapeDtypeStruct((_Q, 1), jnp.float32),
            jax.ShapeDtypeStruct((_Q, 1), jnp.float32),
        ],
    )(dists, meta, query_labels.astype(jnp.int32), query_cameras.astype(jnp.int32))

    mr = jnp.asarray(maxrank, jnp.float32).reshape(1, 1)
    ranks64, map11 = pl.pallas_call(
        _fin_kernel,
        in_specs=[
            pl.BlockSpec((_Q, 1), lambda: (0, 0)),
            pl.BlockSpec((_Q, 1), lambda: (0, 0)),
            pl.BlockSpec((_Q, 1), lambda: (0, 0)),
            pl.BlockSpec((1, 1), lambda: (0, 0)),
        ],
        out_specs=[
            pl.BlockSpec((1, 64), lambda: (0, 0)),
            pl.BlockSpec((1, 1), lambda: (0, 0)),
        ],
        out_shape=[
            jax.ShapeDtypeStruct((1, 64), jnp.float32),
            jax.ShapeDtypeStruct((1, 1), jnp.float32),
        ],
    )(ap, r, cnt, mr)

    return ranks64[0, :_MR], map11[0, 0], dsorted


# speedup vs baseline: 69.2080x; 1.4467x over previous
import jax
import jax.numpy as jnp
from jax.experimental import pallas as pl
from jax.experimental.pallas import tpu as pltpu

_Q, _T, _D = 1024, 16384, 256
_MR = 50
_BQ = 8
_MMB = 128
_BAND = _T // 128


def _key_kernel(q_ref, t_ref, qrn_ref, trn_ref, meta_ref, ql_ref, qc_ref, o_ref):
    acc = jax.lax.dot_general(
        q_ref[...], t_ref[...], (((1,), (1,)), ((), ())),
        preferred_element_type=jnp.float32,
        precision=jax.lax.Precision.DEFAULT)
    d = 1.0 - acc * qrn_ref[...] * trn_ref[...]
    meta = meta_ref[...]
    lab = (meta >> 5) & 127
    cam = (meta >> 2) & 7
    distr = meta & 2
    jnk = meta & 1
    lab_eq = lab == ql_ref[...]
    junk2 = (jnk == 1) | (lab_eq & (cam == qc_ref[...]))
    good = (distr == 0) & jnp.logical_not(junk2) & lab_eq
    b = jax.lax.bitcast_convert_type(d, jnp.int32)
    s = b ^ jnp.where(b < 0, jnp.int32(0x7FFFFFFF), jnp.int32(0))
    o_ref[...] = (s & jnp.int32(~3)) | jnp.where(good, jnp.int32(2), jnp.int32(0)) \
        | jnp.where(junk2, jnp.int32(1), jnp.int32(0))


def _sort_kernel(s_ref, o_ref, ap_ref, fc_ref, fj_ref, cnt_ref):
    g = jax.lax.broadcasted_iota(jnp.int32, (_BAND, 128), 0)
    l = jax.lax.broadcasted_iota(jnp.int32, (_BAND, 128), 1)
    n = (g << 7) | l

    def band(bi, carry):
        row0 = bi * _BAND
        x = s_ref[pl.ds(row0, _BAND), :]

        for m in range(1, _T.bit_length()):
            k = 1 << m
            asc = (n & k) == 0
            j = k // 2
            while j >= 1:
                if j >= 128:
                    t = j >> 7
                    down = pltpu.roll(x, _BAND - t, 0)
                    up = pltpu.roll(x, t, 0)
                    lower = (g & t) == 0
                else:
                    down = pltpu.roll(x, 128 - j, 1)
                    up = pltpu.roll(x, j, 1)
                    lower = (l & j) == 0
                part = jnp.where(lower, down, up)
                keep = (x < part) == (lower == asc)
                x = jnp.where(keep, x, part)
                j >>= 1

        junk_s = x & 1
        good_s = (x >> 1) & 1
        c = (good_s << 16) | junk_s
        for sh in (1, 2, 4, 8, 16, 32, 64):
            c = c + jnp.where(l >= sh, pltpu.roll(c, sh, 1), jnp.int32(0))
        tot = jnp.broadcast_to(jax.lax.slice(c, (0, 127), (_BAND, 128)),
                               (_BAND, 128))
        inc = tot
        sh = 1
        while sh < _BAND:
            inc = inc + jnp.where(g >= sh, pltpu.roll(inc, sh, 0), jnp.int32(0))
            sh *= 2
        c = c + (inc - tot)
        jc = c & 0xFFFF
        gp = c >> 16

        goodb = good_s == 1
        goodf = jnp.where(goodb, 1.0, 0.0).astype(jnp.float32)
        cntp = jnp.sum(goodf, axis=1, keepdims=True)
        terms = gp.astype(jnp.float32) / (n - jc + 1).astype(jnp.float32)
        app = jnp.sum(jnp.where(goodb, terms, 0.0), axis=1, keepdims=True)
        first = goodb & (gp == 1)
        fcp = jnp.sum(jnp.where(first, n, 0), axis=1, keepdims=True).astype(jnp.float32)
        fjp = jnp.sum(jnp.where(first, jc, 0), axis=1, keepdims=True).astype(jnp.float32)

        sc = x & jnp.int32(~3)
        bb = sc ^ jnp.where(sc < 0, jnp.int32(0x7FFFFFFF), jnp.int32(0))
        o_ref[pl.ds(row0, _BAND), :] = jax.lax.bitcast_convert_type(bb, jnp.float32)
        ap_ref[pl.ds(row0, _BAND), :] = app
        fc_ref[pl.ds(row0, _BAND), :] = fcp
        fj_ref[pl.ds(row0, _BAND), :] = fjp
        cnt_ref[pl.ds(row0, _BAND), :] = cntp
        return carry

    jax.lax.fori_loop(0, _BQ, band, 0)


def _fin_kernel(ap_ref, fc_ref, fj_ref, cnt_ref, mr_ref, ranks_ref, map_ref):
    cnt = jnp.sum(cnt_ref[...], axis=1, keepdims=True)
    ap = jnp.sum(ap_ref[...], axis=1, keepdims=True) / jnp.maximum(cnt, 1.0)
    r = (jnp.sum(fc_ref[...], axis=1, keepdims=True)
         - jnp.sum(fj_ref[...], axis=1, keepdims=True))
    valid = (cnt > 0.0) & (r < mr_ref[...])
    cols = jax.lax.broadcasted_iota(jnp.int32, (_Q, 64), 1).astype(jnp.float32)
    hits = jnp.where((r == cols) & valid, 1.0, 0.0)
    hist = jnp.sum(hits, axis=0, keepdims=True)
    iota64 = jax.lax.broadcasted_iota(jnp.int32, (1, 64), 1)
    for sh in (1, 2, 4, 8, 16, 32):
        hist = hist + jnp.where(iota64 >= sh, pltpu.roll(hist, sh, 1), 0.0)
    ranks_ref[...] = hist * (1.0 / _Q)
    map_ref[...] = jnp.sum(ap, axis=0, keepdims=True) * (1.0 / _Q)


def kernel(query_descriptors, test_descriptors, test_labels, test_cameras,
           query_labels, query_cameras, distractors, junk, maxrank):
    qd = query_descriptors.astype(jnp.float32)
    td = test_descriptors.astype(jnp.float32)
    qrn = 1.0 / jnp.sqrt(jnp.sum(qd * qd, axis=1, keepdims=True))
    trn = (1.0 / jnp.sqrt(jnp.sum(td * td, axis=1))).reshape(1, _T)
    meta = ((test_labels.astype(jnp.int32) << 5)
            | (test_cameras.astype(jnp.int32) << 2)
            | (distractors.astype(jnp.int32) << 1)
            | junk.astype(jnp.int32)).reshape(1, _T)

    keys = pl.pallas_call(
        _key_kernel,
        grid=(_Q // _MMB,),
        in_specs=[
            pl.BlockSpec((_MMB, _D), lambda i: (i, 0)),
            pl.BlockSpec((_T, _D), lambda i: (0, 0)),
            pl.BlockSpec((_MMB, 1), lambda i: (i, 0)),
            pl.BlockSpec((1, _T), lambda i: (0, 0)),
            pl.BlockSpec((1, _T), lambda i: (0, 0)),
            pl.BlockSpec((_MMB, 1), lambda i: (i, 0)),
            pl.BlockSpec((_MMB, 1), lambda i: (i, 0)),
        ],
        out_specs=pl.BlockSpec((_MMB, _T), lambda i: (i, 0)),
        out_shape=jax.ShapeDtypeStruct((_Q, _T), jnp.int32),
    )(qd, td, qrn, trn, meta,
      query_labels.astype(jnp.int32), query_cameras.astype(jnp.int32))

    rows = _Q * _BAND
    brows = _BQ * _BAND
    s2 = keys.reshape(rows, 128)
    dsorted2, ap, fc, fj, cnt = pl.pallas_call(
        _sort_kernel,
        grid=(_Q // _BQ,),
        in_specs=[pl.BlockSpec((brows, 128), lambda i: (i, 0))],
        out_specs=[
            pl.BlockSpec((brows, 128), lambda i: (i, 0)),
            pl.BlockSpec((brows, 1), lambda i: (i, 0)),
            pl.BlockSpec((brows, 1), lambda i: (i, 0)),
            pl.BlockSpec((brows, 1), lambda i: (i, 0)),
            pl.BlockSpec((brows, 1), lambda i: (i, 0)),
        ],
        out_shape=[
            jax.ShapeDtypeStruct((rows, 128), jnp.float32),
            jax.ShapeDtypeStruct((rows, 1), jnp.float32),
            jax.ShapeDtypeStruct((rows, 1), jnp.float32),
            jax.ShapeDtypeStruct((rows, 1), jnp.float32),
            jax.ShapeDtypeStruct((rows, 1), jnp.float32),
        ],
    )(s2)
    dsorted = dsorted2.reshape(_Q, _T)

    mr = jnp.asarray(maxrank, jnp.float32).reshape(1, 1)
    ranks64, map11 = pl.pallas_call(
        _fin_kernel,
        in_specs=[
            pl.BlockSpec((_Q, _BAND), lambda: (0, 0)),
            pl.BlockSpec((_Q, _BAND), lambda: (0, 0)),
            pl.BlockSpec((_Q, _BAND), lambda: (0, 0)),
            pl.BlockSpec((_Q, _BAND), lambda: (0, 0)),
            pl.BlockSpec((1, 1), lambda: (0, 0)),
        ],
        out_specs=[
            pl.BlockSpec((1, 64), lambda: (0, 0)),
            pl.BlockSpec((1, 1), lambda: (0, 0)),
        ],
        out_shape=[
            jax.ShapeDtypeStruct((1, 64), jnp.float32),
            jax.ShapeDtypeStruct((1, 1), jnp.float32),
        ],
    )(ap.reshape(_Q, _BAND), fc.reshape(_Q, _BAND),
      fj.reshape(_Q, _BAND), cnt.reshape(_Q, _BAND), mr)

    return ranks64[0, :_MR], map11[0, 0], dsorted


# speedup vs baseline: 98.3072x; 1.4205x over previous
import jax
import jax.numpy as jnp
from jax.experimental import pallas as pl
from jax.experimental.pallas import tpu as pltpu

_Q, _T, _D = 1024, 16384, 256
_MR = 50
_BQ = 2
_MMB = 128
_BAND = _T // 128


def _key_kernel(q_ref, t_ref, qrn_ref, trn_ref, meta_ref, ql_ref, qc_ref, o_ref):
    acc = jax.lax.dot_general(
        q_ref[...], t_ref[...], (((1,), (1,)), ((), ())),
        preferred_element_type=jnp.float32,
        precision=jax.lax.Precision.DEFAULT)
    d = 1.0 - acc * qrn_ref[...] * trn_ref[...]
    meta = meta_ref[...]
    lab = (meta >> 5) & 127
    cam = (meta >> 2) & 7
    distr = meta & 2
    jnk = meta & 1
    lab_eq = lab == ql_ref[...]
    junk2 = (jnk == 1) | (lab_eq & (cam == qc_ref[...]))
    good = (distr == 0) & jnp.logical_not(junk2) & lab_eq
    b = jax.lax.bitcast_convert_type(d, jnp.int32)
    s = b ^ jnp.where(b < 0, jnp.int32(0x7FFFFFFF), jnp.int32(0))
    o_ref[...] = (s & jnp.int32(~3)) | jnp.where(good, jnp.int32(2), jnp.int32(0)) \
        | jnp.where(junk2, jnp.int32(1), jnp.int32(0))


def _sort_kernel(s_ref, o_ref, ap_ref, fc_ref, fj_ref, cnt_ref):
    rows = _BQ * _BAND
    g = jax.lax.broadcasted_iota(jnp.int32, (rows, 128), 0) & (_BAND - 1)
    l = jax.lax.broadcasted_iota(jnp.int32, (rows, 128), 1)
    n = (g << 7) | l
    x = s_ref[...]

    for m in range(1, _T.bit_length()):
        k = 1 << m
        asc = (n & k) == 0
        j = k // 2
        while j >= 1:
            if j >= 128:
                t = j >> 7
                down = pltpu.roll(x, rows - t, 0)
                up = pltpu.roll(x, t, 0)
                lower = (g & t) == 0
            else:
                down = pltpu.roll(x, 128 - j, 1)
                up = pltpu.roll(x, j, 1)
                lower = (l & j) == 0
            part = jnp.where(lower, down, up)
            keep = (x < part) == (lower == asc)
            x = jnp.where(keep, x, part)
            j >>= 1

    junk_s = x & 1
    good_s = (x >> 1) & 1
    c = (good_s << 16) | junk_s
    for sh in (1, 2, 4, 8, 16, 32, 64):
        c = c + jnp.where(l >= sh, pltpu.roll(c, sh, 1), jnp.int32(0))
    tot = jnp.broadcast_to(jax.lax.slice(c, (0, 127), (rows, 128)),
                           (rows, 128))
    inc = tot
    sh = 1
    while sh < _BAND:
        inc = inc + jnp.where(g >= sh, pltpu.roll(inc, sh, 0), jnp.int32(0))
        sh *= 2
    c = c + (inc - tot)
    jc = c & 0xFFFF
    gp = c >> 16

    goodb = good_s == 1
    goodf = jnp.where(goodb, 1.0, 0.0).astype(jnp.float32)
    cnt_ref[...] = jnp.sum(goodf, axis=1, keepdims=True)
    terms = gp.astype(jnp.float32) / (n - jc + 1).astype(jnp.float32)
    ap_ref[...] = jnp.sum(jnp.where(goodb, terms, 0.0), axis=1, keepdims=True)
    first = goodb & (gp == 1)
    fc_ref[...] = jnp.sum(jnp.where(first, n, 0), axis=1,
                          keepdims=True).astype(jnp.float32)
    fj_ref[...] = jnp.sum(jnp.where(first, jc, 0), axis=1,
                          keepdims=True).astype(jnp.float32)

    sc = x & jnp.int32(~3)
    bb = sc ^ jnp.where(sc < 0, jnp.int32(0x7FFFFFFF), jnp.int32(0))
    o_ref[...] = jax.lax.bitcast_convert_type(bb, jnp.float32)


def _fin_kernel(ap_ref, fc_ref, fj_ref, cnt_ref, mr_ref, ranks_ref, map_ref):
    cnt = jnp.sum(cnt_ref[...], axis=1, keepdims=True)
    ap = jnp.sum(ap_ref[...], axis=1, keepdims=True) / jnp.maximum(cnt, 1.0)
    r = (jnp.sum(fc_ref[...], axis=1, keepdims=True)
         - jnp.sum(fj_ref[...], axis=1, keepdims=True))
    valid = (cnt > 0.0) & (r < mr_ref[...])
    cols = jax.lax.broadcasted_iota(jnp.int32, (_Q, 64), 1).astype(jnp.float32)
    hits = jnp.where((r == cols) & valid, 1.0, 0.0)
    hist = jnp.sum(hits, axis=0, keepdims=True)
    iota64 = jax.lax.broadcasted_iota(jnp.int32, (1, 64), 1)
    for sh in (1, 2, 4, 8, 16, 32):
        hist = hist + jnp.where(iota64 >= sh, pltpu.roll(hist, sh, 1), 0.0)
    ranks_ref[...] = hist * (1.0 / _Q)
    map_ref[...] = jnp.sum(ap, axis=0, keepdims=True) * (1.0 / _Q)


def kernel(query_descriptors, test_descriptors, test_labels, test_cameras,
           query_labels, query_cameras, distractors, junk, maxrank):
    qd = query_descriptors.astype(jnp.float32)
    td = test_descriptors.astype(jnp.float32)
    qrn = 1.0 / jnp.sqrt(jnp.sum(qd * qd, axis=1, keepdims=True))
    trn = (1.0 / jnp.sqrt(jnp.sum(td * td, axis=1))).reshape(1, _T)
    meta = ((test_labels.astype(jnp.int32) << 5)
            | (test_cameras.astype(jnp.int32) << 2)
            | (distractors.astype(jnp.int32) << 1)
            | junk.astype(jnp.int32)).reshape(1, _T)

    keys = pl.pallas_call(
        _key_kernel,
        grid=(_Q // _MMB,),
        in_specs=[
            pl.BlockSpec((_MMB, _D), lambda i: (i, 0)),
            pl.BlockSpec((_T, _D), lambda i: (0, 0)),
            pl.BlockSpec((_MMB, 1), lambda i: (i, 0)),
            pl.BlockSpec((1, _T), lambda i: (0, 0)),
            pl.BlockSpec((1, _T), lambda i: (0, 0)),
            pl.BlockSpec((_MMB, 1), lambda i: (i, 0)),
            pl.BlockSpec((_MMB, 1), lambda i: (i, 0)),
        ],
        out_specs=pl.BlockSpec((_MMB, _T), lambda i: (i, 0)),
        out_shape=jax.ShapeDtypeStruct((_Q, _T), jnp.int32),
    )(qd, td, qrn, trn, meta,
      query_labels.astype(jnp.int32), query_cameras.astype(jnp.int32))

    rows = _Q * _BAND
    brows = _BQ * _BAND
    s2 = keys.reshape(rows, 128)
    dsorted2, ap, fc, fj, cnt = pl.pallas_call(
        _sort_kernel,
        grid=(_Q // _BQ,),
        in_specs=[pl.BlockSpec((brows, 128), lambda i: (i, 0))],
        out_specs=[
            pl.BlockSpec((brows, 128), lambda i: (i, 0)),
            pl.BlockSpec((brows, 1), lambda i: (i, 0)),
            pl.BlockSpec((brows, 1), lambda i: (i, 0)),
            pl.BlockSpec((brows, 1), lambda i: (i, 0)),
            pl.BlockSpec((brows, 1), lambda i: (i, 0)),
        ],
        out_shape=[
            jax.ShapeDtypeStruct((rows, 128), jnp.float32),
            jax.ShapeDtypeStruct((rows, 1), jnp.float32),
            jax.ShapeDtypeStruct((rows, 1), jnp.float32),
            jax.ShapeDtypeStruct((rows, 1), jnp.float32),
            jax.ShapeDtypeStruct((rows, 1), jnp.float32),
        ],
    )(s2)
    dsorted = dsorted2.reshape(_Q, _T)

    mr = jnp.asarray(maxrank, jnp.float32).reshape(1, 1)
    ranks64, map11 = pl.pallas_call(
        _fin_kernel,
        in_specs=[
            pl.BlockSpec((_Q, _BAND), lambda: (0, 0)),
            pl.BlockSpec((_Q, _BAND), lambda: (0, 0)),
            pl.BlockSpec((_Q, _BAND), lambda: (0, 0)),
            pl.BlockSpec((_Q, _BAND), lambda: (0, 0)),
            pl.BlockSpec((1, 1), lambda: (0, 0)),
        ],
        out_specs=[
            pl.BlockSpec((1, 64), lambda: (0, 0)),
            pl.BlockSpec((1, 1), lambda: (0, 0)),
        ],
        out_shape=[
            jax.ShapeDtypeStruct((1, 64), jnp.float32),
            jax.ShapeDtypeStruct((1, 1), jnp.float32),
        ],
    )(ap.reshape(_Q, _BAND), fc.reshape(_Q, _BAND),
      fj.reshape(_Q, _BAND), cnt.reshape(_Q, _BAND), mr)

    return ranks64[0, :_MR], map11[0, 0], dsorted


# speedup vs baseline: 155.8730x; 1.5856x over previous
import jax
import jax.numpy as jnp
from jax.experimental import pallas as pl
from jax.experimental.pallas import tpu as pltpu

_Q, _T, _D = 1024, 16384, 256
_MR = 50
_BQ = 2
_MMB = 128
_BAND = _T // 128


def _key_kernel(q_ref, t_ref, qrn_ref, trn_ref, meta_ref, ql_ref, qc_ref, o_ref):
    acc = jax.lax.dot_general(
        q_ref[...], t_ref[...], (((1,), (1,)), ((), ())),
        preferred_element_type=jnp.float32,
        precision=jax.lax.Precision.DEFAULT)
    d = 1.0 - acc * qrn_ref[...] * trn_ref[...]
    meta = meta_ref[...]
    lab = (meta >> 5) & 127
    cam = (meta >> 2) & 7
    distr = meta & 2
    jnk = meta & 1
    lab_eq = lab == ql_ref[...]
    junk2 = (jnk == 1) | (lab_eq & (cam == qc_ref[...]))
    good = (distr == 0) & jnp.logical_not(junk2) & lab_eq
    b = jax.lax.bitcast_convert_type(d, jnp.int32)
    s = b ^ jnp.where(b < 0, jnp.int32(0x7FFFFFFF), jnp.int32(0))
    o_ref[...] = (s & jnp.int32(~3)) | jnp.where(good, jnp.int32(2), jnp.int32(0)) \
        | jnp.where(junk2, jnp.int32(1), jnp.int32(0))


def _sort_kernel(s_ref, o_ref, ap_ref, fc_ref, fj_ref, cnt_ref):
    rows = _BQ * _BAND
    g = jax.lax.broadcasted_iota(jnp.int32, (rows, 128), 0) & (_BAND - 1)
    l = jax.lax.broadcasted_iota(jnp.int32, (rows, 128), 1)
    n = (g << 7) | l
    x = s_ref[...]

    for m in range(1, _T.bit_length()):
        k = 1 << m
        asc = (n & k) == 0
        j = k // 2
        while j >= 1:
            if j >= 128:
                t = j >> 7
                down = pltpu.roll(x, rows - t, 0)
                up = pltpu.roll(x, t, 0)
                lower = (g & t) == 0
                part = jnp.where(lower, down, up)
            else:
                part = jnp.take_along_axis(x, l ^ j, axis=1)
                lower = (l & j) == 0
            keep_min = lower == asc
            x = jnp.where(keep_min, jnp.minimum(x, part), jnp.maximum(x, part))
            j >>= 1

    junk_s = x & 1
    good_s = (x >> 1) & 1
    c = (good_s << 16) | junk_s
    for sh in (1, 2, 4, 8, 16, 32, 64):
        c = c + jnp.where(l >= sh, pltpu.roll(c, sh, 1), jnp.int32(0))
    tot = jnp.broadcast_to(jax.lax.slice(c, (0, 127), (rows, 128)),
                           (rows, 128))
    inc = tot
    sh = 1
    while sh < _BAND:
        inc = inc + jnp.where(g >= sh, pltpu.roll(inc, sh, 0), jnp.int32(0))
        sh *= 2
    c = c + (inc - tot)
    jc = c & 0xFFFF
    gp = c >> 16

    goodb = good_s == 1
    goodf = jnp.where(goodb, 1.0, 0.0).astype(jnp.float32)
    cnt_ref[...] = jnp.sum(goodf, axis=1, keepdims=True)
    terms = gp.astype(jnp.float32) / (n - jc + 1).astype(jnp.float32)
    ap_ref[...] = jnp.sum(jnp.where(goodb, terms, 0.0), axis=1, keepdims=True)
    first = goodb & (gp == 1)
    fc_ref[...] = jnp.sum(jnp.where(first, n, 0), axis=1,
                          keepdims=True).astype(jnp.float32)
    fj_ref[...] = jnp.sum(jnp.where(first, jc, 0), axis=1,
                          keepdims=True).astype(jnp.float32)

    sc = x & jnp.int32(~3)
    bb = sc ^ jnp.where(sc < 0, jnp.int32(0x7FFFFFFF), jnp.int32(0))
    o_ref[...] = jax.lax.bitcast_convert_type(bb, jnp.float32)


def _fin_kernel(ap_ref, fc_ref, fj_ref, cnt_ref, mr_ref, ranks_ref, map_ref):
    cnt = jnp.sum(cnt_ref[...], axis=1, keepdims=True)
    ap = jnp.sum(ap_ref[...], axis=1, keepdims=True) / jnp.maximum(cnt, 1.0)
    r = (jnp.sum(fc_ref[...], axis=1, keepdims=True)
         - jnp.sum(fj_ref[...], axis=1, keepdims=True))
    valid = (cnt > 0.0) & (r < mr_ref[...])
    cols = jax.lax.broadcasted_iota(jnp.int32, (_Q, 64), 1).astype(jnp.float32)
    hits = jnp.where((r == cols) & valid, 1.0, 0.0)
    hist = jnp.sum(hits, axis=0, keepdims=True)
    iota64 = jax.lax.broadcasted_iota(jnp.int32, (1, 64), 1)
    for sh in (1, 2, 4, 8, 16, 32):
        hist = hist + jnp.where(iota64 >= sh, pltpu.roll(hist, sh, 1), 0.0)
    ranks_ref[...] = hist * (1.0 / _Q)
    map_ref[...] = jnp.sum(ap, axis=0, keepdims=True) * (1.0 / _Q)


def kernel(query_descriptors, test_descriptors, test_labels, test_cameras,
           query_labels, query_cameras, distractors, junk, maxrank):
    qd = query_descriptors.astype(jnp.float32)
    td = test_descriptors.astype(jnp.float32)
    qrn = 1.0 / jnp.sqrt(jnp.sum(qd * qd, axis=1, keepdims=True))
    trn = (1.0 / jnp.sqrt(jnp.sum(td * td, axis=1))).reshape(1, _T)
    meta = ((test_labels.astype(jnp.int32) << 5)
            | (test_cameras.astype(jnp.int32) << 2)
            | (distractors.astype(jnp.int32) << 1)
            | junk.astype(jnp.int32)).reshape(1, _T)

    keys = pl.pallas_call(
        _key_kernel,
        grid=(_Q // _MMB,),
        in_specs=[
            pl.BlockSpec((_MMB, _D), lambda i: (i, 0)),
            pl.BlockSpec((_T, _D), lambda i: (0, 0)),
            pl.BlockSpec((_MMB, 1), lambda i: (i, 0)),
            pl.BlockSpec((1, _T), lambda i: (0, 0)),
            pl.BlockSpec((1, _T), lambda i: (0, 0)),
            pl.BlockSpec((_MMB, 1), lambda i: (i, 0)),
            pl.BlockSpec((_MMB, 1), lambda i: (i, 0)),
        ],
        out_specs=pl.BlockSpec((_MMB, _T), lambda i: (i, 0)),
        out_shape=jax.ShapeDtypeStruct((_Q, _T), jnp.int32),
    )(qd, td, qrn, trn, meta,
      query_labels.astype(jnp.int32), query_cameras.astype(jnp.int32))

    rows = _Q * _BAND
    brows = _BQ * _BAND
    s2 = keys.reshape(rows, 128)
    dsorted2, ap, fc, fj, cnt = pl.pallas_call(
        _sort_kernel,
        grid=(_Q // _BQ,),
        in_specs=[pl.BlockSpec((brows, 128), lambda i: (i, 0))],
        out_specs=[
            pl.BlockSpec((brows, 128), lambda i: (i, 0)),
            pl.BlockSpec((brows, 1), lambda i: (i, 0)),
            pl.BlockSpec((brows, 1), lambda i: (i, 0)),
            pl.BlockSpec((brows, 1), lambda i: (i, 0)),
            pl.BlockSpec((brows, 1), lambda i: (i, 0)),
        ],
        out_shape=[
            jax.ShapeDtypeStruct((rows, 128), jnp.float32),
            jax.ShapeDtypeStruct((rows, 1), jnp.float32),
            jax.ShapeDtypeStruct((rows, 1), jnp.float32),
            jax.ShapeDtypeStruct((rows, 1), jnp.float32),
            jax.ShapeDtypeStruct((rows, 1), jnp.float32),
        ],
    )(s2)
    dsorted = dsorted2.reshape(_Q, _T)

    mr = jnp.asarray(maxrank, jnp.float32).reshape(1, 1)
    ranks64, map11 = pl.pallas_call(
        _fin_kernel,
        in_specs=[
            pl.BlockSpec((_Q, _BAND), lambda: (0, 0)),
            pl.BlockSpec((_Q, _BAND), lambda: (0, 0)),
            pl.BlockSpec((_Q, _BAND), lambda: (0, 0)),
            pl.BlockSpec((_Q, _BAND), lambda: (0, 0)),
            pl.BlockSpec((1, 1), lambda: (0, 0)),
        ],
        out_specs=[
            pl.BlockSpec((1, 64), lambda: (0, 0)),
            pl.BlockSpec((1, 1), lambda: (0, 0)),
        ],
        out_shape=[
            jax.ShapeDtypeStruct((1, 64), jnp.float32),
            jax.ShapeDtypeStruct((1, 1), jnp.float32),
        ],
    )(ap.reshape(_Q, _BAND), fc.reshape(_Q, _BAND),
      fj.reshape(_Q, _BAND), cnt.reshape(_Q, _BAND), mr)

    return ranks64[0, :_MR], map11[0, 0], dsorted


# speedup vs baseline: 231.6582x; 1.4862x over previous
import jax
import jax.numpy as jnp
from jax.experimental import pallas as pl
from jax.experimental.pallas import tpu as pltpu

_Q, _T, _D = 1024, 16384, 256
_MR = 50
_BQ = 4
_MMB = 128
_BAND = _T // 128


def _key_kernel(q_ref, t_ref, qrn_ref, trn_ref, meta_ref, ql_ref, qc_ref, o_ref):
    acc = jax.lax.dot_general(
        q_ref[...], t_ref[...], (((1,), (1,)), ((), ())),
        preferred_element_type=jnp.float32,
        precision=jax.lax.Precision.DEFAULT)
    d = 1.0 - acc * qrn_ref[...] * trn_ref[...]
    meta = meta_ref[...]
    lab = (meta >> 5) & 127
    cam = (meta >> 2) & 7
    distr = meta & 2
    jnk = meta & 1
    lab_eq = lab == ql_ref[...]
    junk2 = (jnk == 1) | (lab_eq & (cam == qc_ref[...]))
    good = (distr == 0) & jnp.logical_not(junk2) & lab_eq
    b = jax.lax.bitcast_convert_type(d, jnp.int32)
    s = b ^ jnp.where(b < 0, jnp.int32(0x7FFFFFFF), jnp.int32(0))
    o_ref[...] = (s & jnp.int32(~3)) | jnp.where(good, jnp.int32(2), jnp.int32(0)) \
        | jnp.where(junk2, jnp.int32(1), jnp.int32(0))


def _sort_kernel(s_ref, o_ref, ap_ref, fc_ref, fj_ref, cnt_ref):
    rows = _BQ * _BAND
    g = jax.lax.broadcasted_iota(jnp.int32, (rows, 128), 0) & (_BAND - 1)
    l = jax.lax.broadcasted_iota(jnp.int32, (rows, 128), 1)
    n = (g << 7) | l
    x = s_ref[...]

    for m in range(1, _T.bit_length()):
        k = 1 << m
        asc = (n & k) == 0
        j = k // 2
        while j >= 1:
            if j >= 128:
                t = j >> 7
                down = pltpu.roll(x, rows - t, 0)
                up = pltpu.roll(x, t, 0)
                lower = (g & t) == 0
                part = jnp.where(lower, down, up)
            else:
                part = jnp.take_along_axis(x, l ^ j, axis=1)
                lower = (l & j) == 0
            keep_min = lower == asc
            x = jnp.where(keep_min, jnp.minimum(x, part), jnp.maximum(x, part))
            j >>= 1

    junk_s = x & 1
    good_s = (x >> 1) & 1
    c = (good_s << 16) | junk_s
    for sh in (1, 2, 4, 8, 16, 32, 64):
        c = c + jnp.where(l >= sh, pltpu.roll(c, sh, 1), jnp.int32(0))
    tot = jnp.broadcast_to(jax.lax.slice(c, (0, 127), (rows, 128)),
                           (rows, 128))
    inc = tot
    sh = 1
    while sh < _BAND:
        inc = inc + jnp.where(g >= sh, pltpu.roll(inc, sh, 0), jnp.int32(0))
        sh *= 2
    c = c + (inc - tot)
    jc = c & 0xFFFF
    gp = c >> 16

    goodb = good_s == 1
    goodf = jnp.where(goodb, 1.0, 0.0).astype(jnp.float32)
    cnt_ref[...] = jnp.sum(goodf, axis=1, keepdims=True)
    terms = gp.astype(jnp.float32) / (n - jc + 1).astype(jnp.float32)
    ap_ref[...] = jnp.sum(jnp.where(goodb, terms, 0.0), axis=1, keepdims=True)
    first = goodb & (gp == 1)
    fc_ref[...] = jnp.sum(jnp.where(first, n, 0), axis=1,
                          keepdims=True).astype(jnp.float32)
    fj_ref[...] = jnp.sum(jnp.where(first, jc, 0), axis=1,
                          keepdims=True).astype(jnp.float32)

    sc = x & jnp.int32(~3)
    bb = sc ^ jnp.where(sc < 0, jnp.int32(0x7FFFFFFF), jnp.int32(0))
    o_ref[...] = jax.lax.bitcast_convert_type(bb, jnp.float32)


def _fin_kernel(ap_ref, fc_ref, fj_ref, cnt_ref, mr_ref, ranks_ref, map_ref):
    cnt = jnp.sum(cnt_ref[...], axis=1, keepdims=True)
    ap = jnp.sum(ap_ref[...], axis=1, keepdims=True) / jnp.maximum(cnt, 1.0)
    r = (jnp.sum(fc_ref[...], axis=1, keepdims=True)
         - jnp.sum(fj_ref[...], axis=1, keepdims=True))
    valid = (cnt > 0.0) & (r < mr_ref[...])
    cols = jax.lax.broadcasted_iota(jnp.int32, (_Q, 64), 1).astype(jnp.float32)
    hits = jnp.where((r == cols) & valid, 1.0, 0.0)
    hist = jnp.sum(hits, axis=0, keepdims=True)
    iota64 = jax.lax.broadcasted_iota(jnp.int32, (1, 64), 1)
    for sh in (1, 2, 4, 8, 16, 32):
        hist = hist + jnp.where(iota64 >= sh, pltpu.roll(hist, sh, 1), 0.0)
    ranks_ref[...] = hist * (1.0 / _Q)
    map_ref[...] = jnp.sum(ap, axis=0, keepdims=True) * (1.0 / _Q)


def kernel(query_descriptors, test_descriptors, test_labels, test_cameras,
           query_labels, query_cameras, distractors, junk, maxrank):
    qd = query_descriptors.astype(jnp.float32)
    td = test_descriptors.astype(jnp.float32)
    qrn = 1.0 / jnp.sqrt(jnp.sum(qd * qd, axis=1, keepdims=True))
    trn = (1.0 / jnp.sqrt(jnp.sum(td * td, axis=1))).reshape(1, _T)
    meta = ((test_labels.astype(jnp.int32) << 5)
            | (test_cameras.astype(jnp.int32) << 2)
            | (distractors.astype(jnp.int32) << 1)
            | junk.astype(jnp.int32)).reshape(1, _T)

    keys = pl.pallas_call(
        _key_kernel,
        grid=(_Q // _MMB,),
        in_specs=[
            pl.BlockSpec((_MMB, _D), lambda i: (i, 0)),
            pl.BlockSpec((_T, _D), lambda i: (0, 0)),
            pl.BlockSpec((_MMB, 1), lambda i: (i, 0)),
            pl.BlockSpec((1, _T), lambda i: (0, 0)),
            pl.BlockSpec((1, _T), lambda i: (0, 0)),
            pl.BlockSpec((_MMB, 1), lambda i: (i, 0)),
            pl.BlockSpec((_MMB, 1), lambda i: (i, 0)),
        ],
        out_specs=pl.BlockSpec((_MMB, _T), lambda i: (i, 0)),
        out_shape=jax.ShapeDtypeStruct((_Q, _T), jnp.int32),
    )(qd, td, qrn, trn, meta,
      query_labels.astype(jnp.int32), query_cameras.astype(jnp.int32))

    rows = _Q * _BAND
    brows = _BQ * _BAND
    s2 = keys.reshape(rows, 128)
    dsorted2, ap, fc, fj, cnt = pl.pallas_call(
        _sort_kernel,
        grid=(_Q // _BQ,),
        in_specs=[pl.BlockSpec((brows, 128), lambda i: (i, 0))],
        out_specs=[
            pl.BlockSpec((brows, 128), lambda i: (i, 0)),
            pl.BlockSpec((brows, 1), lambda i: (i, 0)),
            pl.BlockSpec((brows, 1), lambda i: (i, 0)),
            pl.BlockSpec((brows, 1), lambda i: (i, 0)),
            pl.BlockSpec((brows, 1), lambda i: (i, 0)),
        ],
        out_shape=[
            jax.ShapeDtypeStruct((rows, 128), jnp.float32),
            jax.ShapeDtypeStruct((rows, 1), jnp.float32),
            jax.ShapeDtypeStruct((rows, 1), jnp.float32),
            jax.ShapeDtypeStruct((rows, 1), jnp.float32),
            jax.ShapeDtypeStruct((rows, 1), jnp.float32),
        ],
    )(s2)
    dsorted = dsorted2.reshape(_Q, _T)

    mr = jnp.asarray(maxrank, jnp.float32).reshape(1, 1)
    ranks64, map11 = pl.pallas_call(
        _fin_kernel,
        in_specs=[
            pl.BlockSpec((_Q, _BAND), lambda: (0, 0)),
            pl.BlockSpec((_Q, _BAND), lambda: (0, 0)),
            pl.BlockSpec((_Q, _BAND), lambda: (0, 0)),
            pl.BlockSpec((_Q, _BAND), lambda: (0, 0)),
            pl.BlockSpec((1, 1), lambda: (0, 0)),
        ],
        out_specs=[
            pl.BlockSpec((1, 64), lambda: (0, 0)),
            pl.BlockSpec((1, 1), lambda: (0, 0)),
        ],
        out_shape=[
            jax.ShapeDtypeStruct((1, 64), jnp.float32),
            jax.ShapeDtypeStruct((1, 1), jnp.float32),
        ],
    )(ap.reshape(_Q, _BAND), fc.reshape(_Q, _BAND),
      fj.reshape(_Q, _BAND), cnt.reshape(_Q, _BAND), mr)

    return ranks64[0, :_MR], map11[0, 0], dsorted
